# Initial kernel scaffold; baseline (speedup 1.0000x reference)
import jax
import jax.numpy as jnp
from jax.experimental import pallas as pl


def kernel(x_enc, x_mark_enc, token_w, Wq, bq, Wk, bk, Wv, bv, Wo, bo, W1, b1, W2, b2, n1g, n1b, n2g, n2b, fg, fb):
    raise NotImplementedError("write your pallas kernel here")


# speedup vs baseline: 2.8545x; 2.8545x over previous
import numpy as np
import jax
import jax.numpy as jnp
from jax.experimental import pallas as pl
from jax.experimental.pallas import tpu as pltpu

_B, _L, _ENC_IN = 2, 2048, 32
_D, _H, _DFF, _LAYERS = 768, 12, 1024, 2
_DK = _D // _H
_U = 40
_UPAD = 64
_RB = 512
_QB = 256
_PREC = jax.lax.Precision.HIGHEST


def _pos_embedding(length, d_model):
    pe = np.zeros((length, d_model), dtype=np.float32)
    position = np.arange(length, dtype=np.float32)[:, None]
    div_term = np.exp(
        np.arange(0, d_model, 2, dtype=np.float32) * -(np.log(10000.0) / d_model))
    pe[:, 0::2] = np.sin(position * div_term)
    pe[:, 1::2] = np.cos(position * div_term)
    return pe


_POS_PE = _pos_embedding(_L, _D)


def _sample_count_mats():
    base = jax.random.key(42)
    mats = []
    for i in range(_LAYERS):
        lk = jax.random.fold_in(base, i)
        idx = np.asarray(jax.random.randint(lk, (_L, _U), 0, _L))
        c = np.zeros((_L, _L), np.int8)
        np.add.at(c, (np.arange(_L)[:, None], idx), 1)
        mats.append(c)
    return mats


_CMATS = _sample_count_mats()


def _ln(t, g, b):
    mu = jnp.mean(t, axis=-1, keepdims=True)
    var = jnp.mean((t - mu) ** 2, axis=-1, keepdims=True)
    return (t - mu) / jnp.sqrt(var + 1e-5) * g + b


def _dot(a, b):
    return jnp.dot(a, b, preferred_element_type=jnp.float32, precision=_PREC)


def _embed_body(x_ref, w_ref, pe_ref, o_ref):
    o_ref[0] = _dot(x_ref[0], w_ref[...]) + pe_ref[...]


_embed_call = pl.pallas_call(
    _embed_body,
    grid=(_B, _L // _RB),
    in_specs=[
        pl.BlockSpec((1, _RB, 3 * _ENC_IN), lambda b, j: (b, j, 0)),
        pl.BlockSpec((3 * _ENC_IN, _D), lambda b, j: (0, 0)),
        pl.BlockSpec((_RB, _D), lambda b, j: (j, 0)),
    ],
    out_specs=pl.BlockSpec((1, _RB, _D), lambda b, j: (b, j, 0)),
    out_shape=jax.ShapeDtypeStruct((_B, _L, _D), jnp.float32),
)


def _qkv_body(x_ref, wq_ref, wk_ref, wv_ref, bq_ref, bk_ref, bv_ref,
              q_ref, k_ref, v_ref):
    x = x_ref[0]
    q_ref[0] = _dot(x, wq_ref[...]) + bq_ref[...]
    k_ref[0] = _dot(x, wk_ref[...]) + bk_ref[...]
    v_ref[0] = _dot(x, wv_ref[...]) + bv_ref[...]


_qkv_call = pl.pallas_call(
    _qkv_body,
    grid=(_B, _L // _RB),
    in_specs=[
        pl.BlockSpec((1, _RB, _D), lambda b, j: (b, j, 0)),
        pl.BlockSpec((_D, _D), lambda b, j: (0, 0)),
        pl.BlockSpec((_D, _D), lambda b, j: (0, 0)),
        pl.BlockSpec((_D, _D), lambda b, j: (0, 0)),
        pl.BlockSpec((1, _D), lambda b, j: (0, 0)),
        pl.BlockSpec((1, _D), lambda b, j: (0, 0)),
        pl.BlockSpec((1, _D), lambda b, j: (0, 0)),
    ],
    out_specs=[
        pl.BlockSpec((1, _RB, _D), lambda b, j: (b, j, 0)),
        pl.BlockSpec((1, _RB, _D), lambda b, j: (b, j, 0)),
        pl.BlockSpec((1, _RB, _D), lambda b, j: (b, j, 0)),
    ],
    out_shape=[
        jax.ShapeDtypeStruct((_B, _L, _D), jnp.float32),
        jax.ShapeDtypeStruct((_B, _L, _D), jnp.float32),
        jax.ShapeDtypeStruct((_B, _L, _D), jnp.float32),
    ],
)


def _attn_body(c_ref, q_ref, k_ref, v_ref, o_ref,
               m_scr, qred_scr, upd_scr, idx_scr):
    k = k_ref[0, 0]
    v = v_ref[0, 0]
    nblk = _L // _QB

    for j in range(nblk):
        qb = q_ref[0, 0, j * _QB:(j + 1) * _QB, :]
        s = jax.lax.dot_general(qb, k, (((1,), (1,)), ((), ())),
                                preferred_element_type=jnp.float32,
                                precision=_PREC)
        cbf = c_ref[j * _QB:(j + 1) * _QB, :].astype(jnp.float32)
        mx = jnp.max(jnp.where(cbf > 0, s, -jnp.inf), axis=1)
        sm = jnp.sum(s * cbf, axis=1)
        m_scr[j, :] = mx - sm * (1.0 / _L)

    rows = jax.lax.broadcasted_iota(jnp.int32, (nblk, _QB), 0)
    cols = jax.lax.broadcasted_iota(jnp.int32, (nblk, _QB), 1)
    flat = rows * _QB + cols

    def _topk_body(t, m):
        mmax = jnp.max(m)
        pos = jnp.min(jnp.where(m >= mmax, flat, jnp.int32(2 ** 30)))
        idx_scr[t] = pos
        return jnp.where(flat == pos, -jnp.inf, m)

    jax.lax.fori_loop(0, _U, _topk_body, m_scr[...])

    qred_scr[...] = jnp.zeros((_UPAD, _DK), jnp.float32)

    def _gather_body(t, carry):
        p = idx_scr[t]
        qred_scr[pl.ds(t, 1), :] = q_ref[0, 0, pl.ds(p, 1), :]
        return carry

    jax.lax.fori_loop(0, _U, _gather_body, 0)

    scores = jax.lax.dot_general(qred_scr[...], k, (((1,), (1,)), ((), ())),
                                 preferred_element_type=jnp.float32,
                                 precision=_PREC) * (1.0 / np.sqrt(_DK))
    smax = jnp.max(scores, axis=1, keepdims=True)
    e = jnp.exp(scores - smax)
    attn = e / jnp.sum(e, axis=1, keepdims=True)
    upd_scr[...] = jax.lax.dot_general(attn, v, (((1,), (0,)), ((), ())),
                                       preferred_element_type=jnp.float32,
                                       precision=_PREC)

    o_ref[0, 0] = jnp.broadcast_to(jnp.mean(v, axis=0, keepdims=True),
                                   (_L, _DK))

    def _scatter_body(t, carry):
        p = idx_scr[t]
        o_ref[0, 0, pl.ds(p, 1), :] = upd_scr[pl.ds(t, 1), :]
        return carry

    jax.lax.fori_loop(0, _U, _scatter_body, 0)


_attn_call = pl.pallas_call(
    _attn_body,
    grid=(_B, _H),
    in_specs=[
        pl.BlockSpec((_L, _L), lambda b, h: (0, 0)),
        pl.BlockSpec((1, 1, _L, _DK), lambda b, h: (b, h, 0, 0)),
        pl.BlockSpec((1, 1, _L, _DK), lambda b, h: (b, h, 0, 0)),
        pl.BlockSpec((1, 1, _L, _DK), lambda b, h: (b, h, 0, 0)),
    ],
    out_specs=pl.BlockSpec((1, 1, _L, _DK), lambda b, h: (b, h, 0, 0)),
    out_shape=jax.ShapeDtypeStruct((_B, _H, _L, _DK), jnp.float32),
    scratch_shapes=[
        pltpu.VMEM((_L // _QB, _QB), jnp.float32),
        pltpu.VMEM((_UPAD, _DK), jnp.float32),
        pltpu.VMEM((_UPAD, _DK), jnp.float32),
        pltpu.SMEM((_UPAD,), jnp.int32),
    ],
)


def _oproj_body(ctx_ref, enc_ref, w_ref, b_ref, g_ref, bb_ref, o_ref):
    t = _dot(ctx_ref[0], w_ref[...]) + b_ref[...] + enc_ref[0]
    o_ref[0] = _ln(t, g_ref[...], bb_ref[...])


_oproj_call = pl.pallas_call(
    _oproj_body,
    grid=(_B, _L // _RB),
    in_specs=[
        pl.BlockSpec((1, _RB, _D), lambda b, j: (b, j, 0)),
        pl.BlockSpec((1, _RB, _D), lambda b, j: (b, j, 0)),
        pl.BlockSpec((_D, _D), lambda b, j: (0, 0)),
        pl.BlockSpec((1, _D), lambda b, j: (0, 0)),
        pl.BlockSpec((1, _D), lambda b, j: (0, 0)),
        pl.BlockSpec((1, _D), lambda b, j: (0, 0)),
    ],
    out_specs=pl.BlockSpec((1, _RB, _D), lambda b, j: (b, j, 0)),
    out_shape=jax.ShapeDtypeStruct((_B, _L, _D), jnp.float32),
)


def _ffn1_body(x_ref, w_ref, b_ref, o_ref):
    y = _dot(x_ref[0], w_ref[...]) + b_ref[...]
    o_ref[0] = 0.5 * y * (1.0 + jax.lax.erf(y * np.float32(1.0 / np.sqrt(2.0))))


_ffn1_call = pl.pallas_call(
    _ffn1_body,
    grid=(_B, _L // _RB),
    in_specs=[
        pl.BlockSpec((1, _RB, _D), lambda b, j: (b, j, 0)),
        pl.BlockSpec((_D, _DFF), lambda b, j: (0, 0)),
        pl.BlockSpec((1, _DFF), lambda b, j: (0, 0)),
    ],
    out_specs=pl.BlockSpec((1, _RB, _DFF), lambda b, j: (b, j, 0)),
    out_shape=jax.ShapeDtypeStruct((_B, _L, _DFF), jnp.float32),
)


def _ffn2_body(y_ref, xres_ref, w_ref, b_ref, g_ref, bb_ref, o_ref):
    t = _dot(y_ref[0], w_ref[...]) + b_ref[...] + xres_ref[0]
    o_ref[0] = _ln(t, g_ref[...], bb_ref[...])


_ffn2_call = pl.pallas_call(
    _ffn2_body,
    grid=(_B, _L // _RB),
    in_specs=[
        pl.BlockSpec((1, _RB, _DFF), lambda b, j: (b, j, 0)),
        pl.BlockSpec((1, _RB, _D), lambda b, j: (b, j, 0)),
        pl.BlockSpec((_DFF, _D), lambda b, j: (0, 0)),
        pl.BlockSpec((1, _D), lambda b, j: (0, 0)),
        pl.BlockSpec((1, _D), lambda b, j: (0, 0)),
        pl.BlockSpec((1, _D), lambda b, j: (0, 0)),
    ],
    out_specs=pl.BlockSpec((1, _RB, _D), lambda b, j: (b, j, 0)),
    out_shape=jax.ShapeDtypeStruct((_B, _L, _D), jnp.float32),
)


def _final_body(x_ref, mark_ref, g_ref, b_ref, o_ref):
    o_ref[0] = _ln(x_ref[0], g_ref[...], b_ref[...]) * mark_ref[0]


_final_call = pl.pallas_call(
    _final_body,
    grid=(_B, _L // _RB),
    in_specs=[
        pl.BlockSpec((1, _RB, _D), lambda b, j: (b, j, 0)),
        pl.BlockSpec((1, _RB, 1), lambda b, j: (b, j, 0)),
        pl.BlockSpec((1, _D), lambda b, j: (0, 0)),
        pl.BlockSpec((1, _D), lambda b, j: (0, 0)),
    ],
    out_specs=pl.BlockSpec((1, _RB, _D), lambda b, j: (b, j, 0)),
    out_shape=jax.ShapeDtypeStruct((_B, _L, _D), jnp.float32),
)


def kernel(x_enc, x_mark_enc, token_w, Wq, bq, Wk, bk, Wv, bv, Wo, bo,
           W1, b1, W2, b2, n1g, n1b, n2g, n2b, fg, fb):
    xprev = jnp.concatenate([x_enc[:, -1:, :], x_enc[:, :-1, :]], axis=1)
    xnext = jnp.concatenate([x_enc[:, 1:, :], x_enc[:, :1, :]], axis=1)
    xcat = jnp.concatenate([xprev, x_enc, xnext], axis=-1)
    wcat = jnp.transpose(token_w, (2, 1, 0)).reshape(3 * _ENC_IN, _D)
    enc = _embed_call(xcat, wcat, jnp.asarray(_POS_PE))
    mark = x_mark_enc[:, :, None]
    for i in range(_LAYERS):
        q, k, v = _qkv_call(enc, Wq[i], Wk[i], Wv[i],
                            bq[i][None], bk[i][None], bv[i][None])
        qh = q.reshape(_B, _L, _H, _DK).transpose(0, 2, 1, 3)
        kh = k.reshape(_B, _L, _H, _DK).transpose(0, 2, 1, 3)
        vh = v.reshape(_B, _L, _H, _DK).transpose(0, 2, 1, 3)
        ctxh = _attn_call(jnp.asarray(_CMATS[i]), qh, kh, vh)
        ctx = ctxh.transpose(0, 2, 1, 3).reshape(_B, _L, _D)
        xres = _oproj_call(ctx, enc, Wo[i], bo[i][None],
                           n1g[i][None], n1b[i][None])
        y1 = _ffn1_call(xres, W1[i], b1[i][None])
        enc = _ffn2_call(y1, xres, W2[i], b2[i][None],
                         n2g[i][None], n2b[i][None])
    out = _final_call(enc, mark, fg[None], fb[None])
    return out.reshape(_B, _L * _D)


# speedup vs baseline: 4.5866x; 1.6068x over previous
import numpy as np
import jax
import jax.numpy as jnp
from jax.experimental import pallas as pl
from jax.experimental.pallas import tpu as pltpu

_B, _L, _ENC_IN = 2, 2048, 32
_D, _H, _DFF, _LAYERS = 768, 12, 1024, 2
_DK = _D // _H
_U = 40
_UPAD = 64
_RB = 512
_QB = 256
_PREC = jax.lax.Precision.DEFAULT


def _pos_embedding(length, d_model):
    pe = np.zeros((length, d_model), dtype=np.float32)
    position = np.arange(length, dtype=np.float32)[:, None]
    div_term = np.exp(
        np.arange(0, d_model, 2, dtype=np.float32) * -(np.log(10000.0) / d_model))
    pe[:, 0::2] = np.sin(position * div_term)
    pe[:, 1::2] = np.cos(position * div_term)
    return pe


_POS_PE = _pos_embedding(_L, _D)


def _sample_count_mats():
    base = jax.random.key(42)
    mats = []
    for i in range(_LAYERS):
        lk = jax.random.fold_in(base, i)
        idx = np.asarray(jax.random.randint(lk, (_L, _U), 0, _L))
        c = np.zeros((_L, _L), np.int8)
        np.add.at(c, (np.arange(_L)[:, None], idx), 1)
        mats.append(c)
    return mats


_CMATS = _sample_count_mats()


def _ln(t, g, b):
    mu = jnp.mean(t, axis=-1, keepdims=True)
    var = jnp.mean((t - mu) ** 2, axis=-1, keepdims=True)
    return (t - mu) / jnp.sqrt(var + 1e-5) * g + b


def _dot(a, b):
    return jnp.dot(a, b, preferred_element_type=jnp.float32, precision=_PREC)


def _embed_body(x_ref, w_ref, pe_ref, o_ref):
    o_ref[0] = _dot(x_ref[0], w_ref[...]) + pe_ref[...]


_embed_call = pl.pallas_call(
    _embed_body,
    grid=(_B, _L // _RB),
    in_specs=[
        pl.BlockSpec((1, _RB, 3 * _ENC_IN), lambda b, j: (b, j, 0)),
        pl.BlockSpec((3 * _ENC_IN, _D), lambda b, j: (0, 0)),
        pl.BlockSpec((_RB, _D), lambda b, j: (j, 0)),
    ],
    out_specs=pl.BlockSpec((1, _RB, _D), lambda b, j: (b, j, 0)),
    out_shape=jax.ShapeDtypeStruct((_B, _L, _D), jnp.float32),
)


def _qkv_body(x_ref, wq_ref, wk_ref, wv_ref, bq_ref, bk_ref, bv_ref,
              q_ref, k_ref, v_ref):
    x = x_ref[0]
    q_ref[0] = _dot(x, wq_ref[...]) + bq_ref[...]
    k_ref[0] = _dot(x, wk_ref[...]) + bk_ref[...]
    v_ref[0] = _dot(x, wv_ref[...]) + bv_ref[...]


_qkv_call = pl.pallas_call(
    _qkv_body,
    grid=(_B, _L // _RB),
    in_specs=[
        pl.BlockSpec((1, _RB, _D), lambda b, j: (b, j, 0)),
        pl.BlockSpec((_D, _D), lambda b, j: (0, 0)),
        pl.BlockSpec((_D, _D), lambda b, j: (0, 0)),
        pl.BlockSpec((_D, _D), lambda b, j: (0, 0)),
        pl.BlockSpec((1, _D), lambda b, j: (0, 0)),
        pl.BlockSpec((1, _D), lambda b, j: (0, 0)),
        pl.BlockSpec((1, _D), lambda b, j: (0, 0)),
    ],
    out_specs=[
        pl.BlockSpec((1, _RB, _D), lambda b, j: (b, j, 0)),
        pl.BlockSpec((1, _RB, _D), lambda b, j: (b, j, 0)),
        pl.BlockSpec((1, _RB, _D), lambda b, j: (b, j, 0)),
    ],
    out_shape=[
        jax.ShapeDtypeStruct((_B, _L, _D), jnp.float32),
        jax.ShapeDtypeStruct((_B, _L, _D), jnp.float32),
        jax.ShapeDtypeStruct((_B, _L, _D), jnp.float32),
    ],
)


def _attn_body(c_ref, q_ref, k_ref, v_ref, o_ref,
               m_scr, qred_scr, upd_scr, idx_scr):
    k = k_ref[0, 0]
    v = v_ref[0, 0]
    nblk = _L // _QB

    for j in range(nblk):
        qb = q_ref[0, 0, j * _QB:(j + 1) * _QB, :]
        s = jax.lax.dot_general(qb, k, (((1,), (1,)), ((), ())),
                                preferred_element_type=jnp.float32,
                                precision=_PREC)
        cbf = c_ref[j * _QB:(j + 1) * _QB, :].astype(jnp.float32)
        mx = jnp.max(jnp.where(cbf > 0, s, -jnp.inf), axis=1)
        sm = jnp.sum(s * cbf, axis=1)
        m_scr[j, :] = mx - sm * (1.0 / _L)

    rows = jax.lax.broadcasted_iota(jnp.int32, (nblk, _QB), 0)
    cols = jax.lax.broadcasted_iota(jnp.int32, (nblk, _QB), 1)
    flat = rows * _QB + cols

    def _topk_body(t, m):
        mmax = jnp.max(m)
        pos = jnp.min(jnp.where(m >= mmax, flat, jnp.int32(2 ** 30)))
        idx_scr[t] = pos
        return jnp.where(flat == pos, -jnp.inf, m)

    jax.lax.fori_loop(0, _U, _topk_body, m_scr[...])

    qred_scr[...] = jnp.zeros((_UPAD, _DK), jnp.float32)

    def _gather_body(t, carry):
        p = idx_scr[t]
        qred_scr[pl.ds(t, 1), :] = q_ref[0, 0, pl.ds(p, 1), :]
        return carry

    jax.lax.fori_loop(0, _U, _gather_body, 0)

    scores = jax.lax.dot_general(qred_scr[...], k, (((1,), (1,)), ((), ())),
                                 preferred_element_type=jnp.float32,
                                 precision=_PREC) * (1.0 / np.sqrt(_DK))
    smax = jnp.max(scores, axis=1, keepdims=True)
    e = jnp.exp(scores - smax)
    attn = e / jnp.sum(e, axis=1, keepdims=True)
    upd_scr[...] = jax.lax.dot_general(attn, v, (((1,), (0,)), ((), ())),
                                       preferred_element_type=jnp.float32,
                                       precision=_PREC)

    o_ref[0, 0] = jnp.broadcast_to(jnp.mean(v, axis=0, keepdims=True),
                                   (_L, _DK))

    def _scatter_body(t, carry):
        p = idx_scr[t]
        o_ref[0, 0, pl.ds(p, 1), :] = upd_scr[pl.ds(t, 1), :]
        return carry

    jax.lax.fori_loop(0, _U, _scatter_body, 0)


_attn_call = pl.pallas_call(
    _attn_body,
    grid=(_B, _H),
    in_specs=[
        pl.BlockSpec((_L, _L), lambda b, h: (0, 0)),
        pl.BlockSpec((1, 1, _L, _DK), lambda b, h: (b, h, 0, 0)),
        pl.BlockSpec((1, 1, _L, _DK), lambda b, h: (b, h, 0, 0)),
        pl.BlockSpec((1, 1, _L, _DK), lambda b, h: (b, h, 0, 0)),
    ],
    out_specs=pl.BlockSpec((1, 1, _L, _DK), lambda b, h: (b, h, 0, 0)),
    out_shape=jax.ShapeDtypeStruct((_B, _H, _L, _DK), jnp.float32),
    scratch_shapes=[
        pltpu.VMEM((_L // _QB, _QB), jnp.float32),
        pltpu.VMEM((_UPAD, _DK), jnp.float32),
        pltpu.VMEM((_UPAD, _DK), jnp.float32),
        pltpu.SMEM((_UPAD,), jnp.int32),
    ],
)


def _oproj_body(ctx_ref, enc_ref, w_ref, b_ref, g_ref, bb_ref, o_ref):
    t = _dot(ctx_ref[0], w_ref[...]) + b_ref[...] + enc_ref[0]
    o_ref[0] = _ln(t, g_ref[...], bb_ref[...])


_oproj_call = pl.pallas_call(
    _oproj_body,
    grid=(_B, _L // _RB),
    in_specs=[
        pl.BlockSpec((1, _RB, _D), lambda b, j: (b, j, 0)),
        pl.BlockSpec((1, _RB, _D), lambda b, j: (b, j, 0)),
        pl.BlockSpec((_D, _D), lambda b, j: (0, 0)),
        pl.BlockSpec((1, _D), lambda b, j: (0, 0)),
        pl.BlockSpec((1, _D), lambda b, j: (0, 0)),
        pl.BlockSpec((1, _D), lambda b, j: (0, 0)),
    ],
    out_specs=pl.BlockSpec((1, _RB, _D), lambda b, j: (b, j, 0)),
    out_shape=jax.ShapeDtypeStruct((_B, _L, _D), jnp.float32),
)


def _ffn1_body(x_ref, w_ref, b_ref, o_ref):
    y = _dot(x_ref[0], w_ref[...]) + b_ref[...]
    o_ref[0] = 0.5 * y * (1.0 + jax.lax.erf(y * np.float32(1.0 / np.sqrt(2.0))))


_ffn1_call = pl.pallas_call(
    _ffn1_body,
    grid=(_B, _L // _RB),
    in_specs=[
        pl.BlockSpec((1, _RB, _D), lambda b, j: (b, j, 0)),
        pl.BlockSpec((_D, _DFF), lambda b, j: (0, 0)),
        pl.BlockSpec((1, _DFF), lambda b, j: (0, 0)),
    ],
    out_specs=pl.BlockSpec((1, _RB, _DFF), lambda b, j: (b, j, 0)),
    out_shape=jax.ShapeDtypeStruct((_B, _L, _DFF), jnp.float32),
)


def _ffn2_body(y_ref, xres_ref, w_ref, b_ref, g_ref, bb_ref, o_ref):
    t = _dot(y_ref[0], w_ref[...]) + b_ref[...] + xres_ref[0]
    o_ref[0] = _ln(t, g_ref[...], bb_ref[...])


_ffn2_call = pl.pallas_call(
    _ffn2_body,
    grid=(_B, _L // _RB),
    in_specs=[
        pl.BlockSpec((1, _RB, _DFF), lambda b, j: (b, j, 0)),
        pl.BlockSpec((1, _RB, _D), lambda b, j: (b, j, 0)),
        pl.BlockSpec((_DFF, _D), lambda b, j: (0, 0)),
        pl.BlockSpec((1, _D), lambda b, j: (0, 0)),
        pl.BlockSpec((1, _D), lambda b, j: (0, 0)),
        pl.BlockSpec((1, _D), lambda b, j: (0, 0)),
    ],
    out_specs=pl.BlockSpec((1, _RB, _D), lambda b, j: (b, j, 0)),
    out_shape=jax.ShapeDtypeStruct((_B, _L, _D), jnp.float32),
)


def _final_body(x_ref, mark_ref, g_ref, b_ref, o_ref):
    o_ref[0] = _ln(x_ref[0], g_ref[...], b_ref[...]) * mark_ref[0]


_final_call = pl.pallas_call(
    _final_body,
    grid=(_B, _L // _RB),
    in_specs=[
        pl.BlockSpec((1, _RB, _D), lambda b, j: (b, j, 0)),
        pl.BlockSpec((1, _RB, 1), lambda b, j: (b, j, 0)),
        pl.BlockSpec((1, _D), lambda b, j: (0, 0)),
        pl.BlockSpec((1, _D), lambda b, j: (0, 0)),
    ],
    out_specs=pl.BlockSpec((1, _RB, _D), lambda b, j: (b, j, 0)),
    out_shape=jax.ShapeDtypeStruct((_B, _L, _D), jnp.float32),
)


def kernel(x_enc, x_mark_enc, token_w, Wq, bq, Wk, bk, Wv, bv, Wo, bo,
           W1, b1, W2, b2, n1g, n1b, n2g, n2b, fg, fb):
    xprev = jnp.concatenate([x_enc[:, -1:, :], x_enc[:, :-1, :]], axis=1)
    xnext = jnp.concatenate([x_enc[:, 1:, :], x_enc[:, :1, :]], axis=1)
    xcat = jnp.concatenate([xprev, x_enc, xnext], axis=-1)
    wcat = jnp.transpose(token_w, (2, 1, 0)).reshape(3 * _ENC_IN, _D)
    enc = _embed_call(xcat, wcat, jnp.asarray(_POS_PE))
    mark = x_mark_enc[:, :, None]
    for i in range(_LAYERS):
        q, k, v = _qkv_call(enc, Wq[i], Wk[i], Wv[i],
                            bq[i][None], bk[i][None], bv[i][None])
        qh = q.reshape(_B, _L, _H, _DK).transpose(0, 2, 1, 3)
        kh = k.reshape(_B, _L, _H, _DK).transpose(0, 2, 1, 3)
        vh = v.reshape(_B, _L, _H, _DK).transpose(0, 2, 1, 3)
        ctxh = _attn_call(jnp.asarray(_CMATS[i]), qh, kh, vh)
        ctx = ctxh.transpose(0, 2, 1, 3).reshape(_B, _L, _D)
        xres = _oproj_call(ctx, enc, Wo[i], bo[i][None],
                           n1g[i][None], n1b[i][None])
        y1 = _ffn1_call(xres, W1[i], b1[i][None])
        enc = _ffn2_call(y1, xres, W2[i], b2[i][None],
                         n2g[i][None], n2b[i][None])
    out = _final_call(enc, mark, fg[None], fb[None])
    return out.reshape(_B, _L * _D)


# speedup vs baseline: 9.3185x; 2.0317x over previous
import numpy as np
import jax
import jax.numpy as jnp
from jax.experimental import pallas as pl
from jax.experimental.pallas import tpu as pltpu

_B, _L, _ENC_IN = 2, 2048, 32
_D, _H, _DFF, _LAYERS = 768, 12, 1024, 2
_DK = _D // _H
_HP = _H // 2
_U = 40
_UPAD = 64
_RB = 512
_QB = 256
_PREC = jax.lax.Precision.DEFAULT


def _pos_embedding(length, d_model):
    pe = np.zeros((length, d_model), dtype=np.float32)
    position = np.arange(length, dtype=np.float32)[:, None]
    div_term = np.exp(
        np.arange(0, d_model, 2, dtype=np.float32) * -(np.log(10000.0) / d_model))
    pe[:, 0::2] = np.sin(position * div_term)
    pe[:, 1::2] = np.cos(position * div_term)
    return pe


_POS_PE = _pos_embedding(_L, _D)


def _sample_count_mats():
    base = jax.random.key(42)
    mats = []
    for i in range(_LAYERS):
        lk = jax.random.fold_in(base, i)
        idx = np.asarray(jax.random.randint(lk, (_L, _U), 0, _L))
        c = np.zeros((_L, _L), np.float32)
        np.add.at(c, (np.arange(_L)[:, None], idx), 1.0)
        mats.append(c)
    return mats


_CMATS = _sample_count_mats()


def _ln(t, g, b):
    mu = jnp.mean(t, axis=-1, keepdims=True)
    var = jnp.mean((t - mu) ** 2, axis=-1, keepdims=True)
    return (t - mu) / jnp.sqrt(var + 1e-5) * g + b


def _dot(a, b):
    return jnp.dot(a, b, preferred_element_type=jnp.float32, precision=_PREC)


def _dg(a, b, dims):
    return jax.lax.dot_general(a, b, (dims, ((), ())),
                               preferred_element_type=jnp.float32,
                               precision=_PREC)


def _embed_body(x_ref, w_ref, pe_ref, o_ref):
    o_ref[0] = _dot(x_ref[0], w_ref[...]) + pe_ref[...]


_embed_call = pl.pallas_call(
    _embed_body,
    grid=(_B, _L // _RB),
    in_specs=[
        pl.BlockSpec((1, _RB, 3 * _ENC_IN), lambda b, j: (b, j, 0)),
        pl.BlockSpec((3 * _ENC_IN, _D), lambda b, j: (0, 0)),
        pl.BlockSpec((_RB, _D), lambda b, j: (j, 0)),
    ],
    out_specs=pl.BlockSpec((1, _RB, _D), lambda b, j: (b, j, 0)),
    out_shape=jax.ShapeDtypeStruct((_B, _L, _D), jnp.float32),
)


def _qkv_body(x_ref, wq_ref, wk_ref, wv_ref, bq_ref, bk_ref, bv_ref,
              q_ref, k_ref, v_ref):
    x = x_ref[0]
    q_ref[0] = _dot(x, wq_ref[...]) + bq_ref[...]
    k_ref[0] = _dot(x, wk_ref[...]) + bk_ref[...]
    v_ref[0] = _dot(x, wv_ref[...]) + bv_ref[...]


_qkv_call = pl.pallas_call(
    _qkv_body,
    grid=(_B, _L // _RB),
    in_specs=[
        pl.BlockSpec((1, _RB, _D), lambda b, j: (b, j, 0)),
        pl.BlockSpec((_D, _D), lambda b, j: (0, 0)),
        pl.BlockSpec((_D, _D), lambda b, j: (0, 0)),
        pl.BlockSpec((_D, _D), lambda b, j: (0, 0)),
        pl.BlockSpec((1, _D), lambda b, j: (0, 0)),
        pl.BlockSpec((1, _D), lambda b, j: (0, 0)),
        pl.BlockSpec((1, _D), lambda b, j: (0, 0)),
    ],
    out_specs=[
        pl.BlockSpec((1, _RB, _D), lambda b, j: (b, j, 0)),
        pl.BlockSpec((1, _RB, _D), lambda b, j: (b, j, 0)),
        pl.BlockSpec((1, _RB, _D), lambda b, j: (b, j, 0)),
    ],
    out_shape=[
        jax.ShapeDtypeStruct((_B, _L, _D), jnp.float32),
        jax.ShapeDtypeStruct((_B, _L, _D), jnp.float32),
        jax.ShapeDtypeStruct((_B, _L, _D), jnp.float32),
    ],
)


def _m_body(c_ref, q_ref, k_ref, o_ref, m_scr):
    nblk = _L // _QB
    kcp = jax.lax.dot_general(
        c_ref[...], k_ref[0].astype(jnp.bfloat16), (((1,), (0,)), ((), ())),
        preferred_element_type=jnp.float32)
    for hh in range(2):
        qh = q_ref[0][:, hh * _DK:(hh + 1) * _DK]
        kh = k_ref[0][:, hh * _DK:(hh + 1) * _DK]
        kch = kcp[:, hh * _DK:(hh + 1) * _DK]
        for j in range(nblk):
            qb = qh[j * _QB:(j + 1) * _QB, :]
            s = _dg(qb, kh, ((1,), (1,)))
            cb = c_ref[j * _QB:(j + 1) * _QB, :]
            mx = jnp.max(jnp.where(cb > 0, s, -jnp.inf), axis=1)
            sm = jnp.sum(qb * kch[j * _QB:(j + 1) * _QB, :], axis=1)
            m_scr[j, :] = mx - sm * (1.0 / _L)
        o_ref[0, hh] = jnp.concatenate(
            [m_scr[j:j + 1, :] for j in range(nblk)], axis=1)


_m_call = pl.pallas_call(
    _m_body,
    grid=(_B, _HP),
    in_specs=[
        pl.BlockSpec((_L, _L), lambda b, p: (0, 0)),
        pl.BlockSpec((1, _L, 2 * _DK), lambda b, p: (b, 0, p)),
        pl.BlockSpec((1, _L, 2 * _DK), lambda b, p: (b, 0, p)),
    ],
    out_specs=pl.BlockSpec((1, 2, 1, _L), lambda b, p: (b, p, 0, 0)),
    out_shape=jax.ShapeDtypeStruct((_B, _H, 1, _L), jnp.float32),
    scratch_shapes=[
        pltpu.VMEM((_L // _QB, _QB), jnp.float32),
    ],
)


def _topk_body(m_ref, r_ref):
    m = m_ref[...]
    lanes = jax.lax.broadcasted_iota(jnp.int32, (_B * _H, _L), 1)
    rank = jnp.full((_B * _H, _L), -1, jnp.int32)
    big = jnp.int32(2 ** 30)
    for t in range(_U):
        rmax = jnp.max(m, axis=1, keepdims=True)
        cand = jnp.where(m >= rmax, lanes, big)
        pos = jnp.min(cand, axis=1, keepdims=True)
        oh = lanes == pos
        rank = jnp.where(oh, jnp.int32(t), rank)
        m = jnp.where(oh, -jnp.inf, m)
    r_ref[...] = rank


_topk_call = pl.pallas_call(
    _topk_body,
    out_shape=jax.ShapeDtypeStruct((_B * _H, _L), jnp.int32),
)


def _apply_body(r_ref, q_ref, k_ref, v_ref, o_ref):
    outs = []
    riota = jax.lax.broadcasted_iota(jnp.int32, (_UPAD, _L), 0)
    for hh in range(2):
        rank_flat = r_ref[0, hh]
        q = q_ref[0][:, hh * _DK:(hh + 1) * _DK]
        k = k_ref[0][:, hh * _DK:(hh + 1) * _DK]
        v = v_ref[0][:, hh * _DK:(hh + 1) * _DK]
        p_mat = (riota == rank_flat).astype(jnp.float32)
        qred = _dg(p_mat, q, ((1,), (0,)))
        scores = _dg(qred, k, ((1,), (1,))) * (1.0 / np.sqrt(_DK))
        smax = jnp.max(scores, axis=1, keepdims=True)
        e = jnp.exp(scores - smax)
        attn = e / jnp.sum(e, axis=1, keepdims=True)
        upd = _dg(attn, v, ((1,), (0,)))
        sel_col = _dg(p_mat, jnp.ones((_UPAD, 1), jnp.float32),
                      ((0,), (0,)))
        upd_rows = _dg(p_mat, upd, ((0,), (0,)))
        vmean = jnp.mean(v, axis=0, keepdims=True)
        outs.append(jnp.broadcast_to(vmean, (_L, _DK)) * (1.0 - sel_col)
                    + upd_rows)
    o_ref[0] = jnp.concatenate(outs, axis=1)


_apply_call = pl.pallas_call(
    _apply_body,
    grid=(_B, _HP),
    in_specs=[
        pl.BlockSpec((1, 2, 1, _L), lambda b, p: (b, p, 0, 0)),
        pl.BlockSpec((1, _L, 2 * _DK), lambda b, p: (b, 0, p)),
        pl.BlockSpec((1, _L, 2 * _DK), lambda b, p: (b, 0, p)),
        pl.BlockSpec((1, _L, 2 * _DK), lambda b, p: (b, 0, p)),
    ],
    out_specs=pl.BlockSpec((1, _L, 2 * _DK), lambda b, p: (b, 0, p)),
    out_shape=jax.ShapeDtypeStruct((_B, _L, _D), jnp.float32),
)


def _oproj_body(ctx_ref, enc_ref, w_ref, b_ref, g_ref, bb_ref, o_ref):
    t = _dot(ctx_ref[0], w_ref[...]) + b_ref[...] + enc_ref[0]
    o_ref[0] = _ln(t, g_ref[...], bb_ref[...])


_oproj_call = pl.pallas_call(
    _oproj_body,
    grid=(_B, _L // _RB),
    in_specs=[
        pl.BlockSpec((1, _RB, _D), lambda b, j: (b, j, 0)),
        pl.BlockSpec((1, _RB, _D), lambda b, j: (b, j, 0)),
        pl.BlockSpec((_D, _D), lambda b, j: (0, 0)),
        pl.BlockSpec((1, _D), lambda b, j: (0, 0)),
        pl.BlockSpec((1, _D), lambda b, j: (0, 0)),
        pl.BlockSpec((1, _D), lambda b, j: (0, 0)),
    ],
    out_specs=pl.BlockSpec((1, _RB, _D), lambda b, j: (b, j, 0)),
    out_shape=jax.ShapeDtypeStruct((_B, _L, _D), jnp.float32),
)


def _ffn1_body(x_ref, w_ref, b_ref, o_ref):
    y = _dot(x_ref[0], w_ref[...]) + b_ref[...]
    o_ref[0] = 0.5 * y * (1.0 + jax.lax.erf(y * np.float32(1.0 / np.sqrt(2.0))))


_ffn1_call = pl.pallas_call(
    _ffn1_body,
    grid=(_B, _L // _RB),
    in_specs=[
        pl.BlockSpec((1, _RB, _D), lambda b, j: (b, j, 0)),
        pl.BlockSpec((_D, _DFF), lambda b, j: (0, 0)),
        pl.BlockSpec((1, _DFF), lambda b, j: (0, 0)),
    ],
    out_specs=pl.BlockSpec((1, _RB, _DFF), lambda b, j: (b, j, 0)),
    out_shape=jax.ShapeDtypeStruct((_B, _L, _DFF), jnp.float32),
)


def _ffn2_body(y_ref, xres_ref, w_ref, b_ref, g_ref, bb_ref, o_ref):
    t = _dot(y_ref[0], w_ref[...]) + b_ref[...] + xres_ref[0]
    o_ref[0] = _ln(t, g_ref[...], bb_ref[...])


_ffn2_call = pl.pallas_call(
    _ffn2_body,
    grid=(_B, _L // _RB),
    in_specs=[
        pl.BlockSpec((1, _RB, _DFF), lambda b, j: (b, j, 0)),
        pl.BlockSpec((1, _RB, _D), lambda b, j: (b, j, 0)),
        pl.BlockSpec((_DFF, _D), lambda b, j: (0, 0)),
        pl.BlockSpec((1, _D), lambda b, j: (0, 0)),
        pl.BlockSpec((1, _D), lambda b, j: (0, 0)),
        pl.BlockSpec((1, _D), lambda b, j: (0, 0)),
    ],
    out_specs=pl.BlockSpec((1, _RB, _D), lambda b, j: (b, j, 0)),
    out_shape=jax.ShapeDtypeStruct((_B, _L, _D), jnp.float32),
)


def _final_body(x_ref, mark_ref, g_ref, b_ref, o_ref):
    o_ref[0] = _ln(x_ref[0], g_ref[...], b_ref[...]) * mark_ref[0]


_final_call = pl.pallas_call(
    _final_body,
    grid=(_B, _L // _RB),
    in_specs=[
        pl.BlockSpec((1, _RB, _D), lambda b, j: (b, j, 0)),
        pl.BlockSpec((1, _RB, 1), lambda b, j: (b, j, 0)),
        pl.BlockSpec((1, _D), lambda b, j: (0, 0)),
        pl.BlockSpec((1, _D), lambda b, j: (0, 0)),
    ],
    out_specs=pl.BlockSpec((1, _RB, _D), lambda b, j: (b, j, 0)),
    out_shape=jax.ShapeDtypeStruct((_B, _L, _D), jnp.float32),
)


def kernel(x_enc, x_mark_enc, token_w, Wq, bq, Wk, bk, Wv, bv, Wo, bo,
           W1, b1, W2, b2, n1g, n1b, n2g, n2b, fg, fb):
    xprev = jnp.concatenate([x_enc[:, -1:, :], x_enc[:, :-1, :]], axis=1)
    xnext = jnp.concatenate([x_enc[:, 1:, :], x_enc[:, :1, :]], axis=1)
    xcat = jnp.concatenate([xprev, x_enc, xnext], axis=-1)
    wcat = jnp.transpose(token_w, (2, 1, 0)).reshape(3 * _ENC_IN, _D)
    enc = _embed_call(xcat, wcat, jnp.asarray(_POS_PE))
    mark = x_mark_enc[:, :, None]
    for i in range(_LAYERS):
        cmat = jnp.asarray(_CMATS[i], dtype=jnp.bfloat16)
        q, k, v = _qkv_call(enc, Wq[i], Wk[i], Wv[i],
                            bq[i][None], bk[i][None], bv[i][None])
        m = _m_call(cmat, q, k)
        rank = _topk_call(m.reshape(_B * _H, _L))
        ctx = _apply_call(rank.reshape(_B, _H, 1, _L), q, k, v)
        xres = _oproj_call(ctx, enc, Wo[i], bo[i][None],
                           n1g[i][None], n1b[i][None])
        y1 = _ffn1_call(xres, W1[i], b1[i][None])
        enc = _ffn2_call(y1, xres, W2[i], b2[i][None],
                         n2g[i][None], n2b[i][None])
    out = _final_call(enc, mark, fg[None], fb[None])
    return out.reshape(_B, _L * _D)


# speedup vs baseline: 9.5817x; 1.0282x over previous
import numpy as np
import jax
import jax.numpy as jnp
from jax.experimental import pallas as pl
from jax.experimental.pallas import tpu as pltpu

_B, _L, _ENC_IN = 2, 2048, 32
_D, _H, _DFF, _LAYERS = 768, 12, 1024, 2
_DK = _D // _H
_HP = _H // 2
_U = 40
_UPAD = 64
_RB = 512
_QB = 256
_PREC = jax.lax.Precision.DEFAULT


def _pos_embedding(length, d_model):
    pe = np.zeros((length, d_model), dtype=np.float32)
    position = np.arange(length, dtype=np.float32)[:, None]
    div_term = np.exp(
        np.arange(0, d_model, 2, dtype=np.float32) * -(np.log(10000.0) / d_model))
    pe[:, 0::2] = np.sin(position * div_term)
    pe[:, 1::2] = np.cos(position * div_term)
    return pe


_POS_PE = _pos_embedding(_L, _D)


def _sample_count_mats():
    base = jax.random.key(42)
    mats = []
    for i in range(_LAYERS):
        lk = jax.random.fold_in(base, i)
        idx = np.asarray(jax.random.randint(lk, (_L, _U), 0, _L))
        c = np.zeros((_L, _L), np.float32)
        np.add.at(c, (np.arange(_L)[:, None], idx), 1.0)
        mats.append(c)
    return mats


_CMATS = _sample_count_mats()
_MASKNEG = [np.where(c > 0, np.float32(0), np.float32(-np.inf))
            for c in _CMATS]


def _ln(t, g, b):
    mu = jnp.mean(t, axis=-1, keepdims=True)
    var = jnp.mean((t - mu) ** 2, axis=-1, keepdims=True)
    return (t - mu) / jnp.sqrt(var + 1e-5) * g + b


def _dot(a, b):
    return jnp.dot(a, b, preferred_element_type=jnp.float32, precision=_PREC)


def _dg(a, b, dims):
    return jax.lax.dot_general(a, b, (dims, ((), ())),
                               preferred_element_type=jnp.float32,
                               precision=_PREC)


def _embed_body(x_ref, w_ref, pe_ref, o_ref):
    o_ref[0] = _dot(x_ref[0], w_ref[...]) + pe_ref[...]


_embed_call = pl.pallas_call(
    _embed_body,
    grid=(_B, _L // _RB),
    in_specs=[
        pl.BlockSpec((1, _RB, 3 * _ENC_IN), lambda b, j: (b, j, 0)),
        pl.BlockSpec((3 * _ENC_IN, _D), lambda b, j: (0, 0)),
        pl.BlockSpec((_RB, _D), lambda b, j: (j, 0)),
    ],
    out_specs=pl.BlockSpec((1, _RB, _D), lambda b, j: (b, j, 0)),
    out_shape=jax.ShapeDtypeStruct((_B, _L, _D), jnp.float32),
)


def _qkv_body(x_ref, wq_ref, wk_ref, wv_ref, bq_ref, bk_ref, bv_ref,
              q_ref, k_ref, v_ref):
    x = x_ref[0]
    q_ref[0] = _dot(x, wq_ref[...]) + bq_ref[...]
    k_ref[0] = _dot(x, wk_ref[...]) + bk_ref[...]
    v_ref[0] = _dot(x, wv_ref[...]) + bv_ref[...]


_qkv_call = pl.pallas_call(
    _qkv_body,
    grid=(_B, _L // _RB),
    in_specs=[
        pl.BlockSpec((1, _RB, _D), lambda b, j: (b, j, 0)),
        pl.BlockSpec((_D, _D), lambda b, j: (0, 0)),
        pl.BlockSpec((_D, _D), lambda b, j: (0, 0)),
        pl.BlockSpec((_D, _D), lambda b, j: (0, 0)),
        pl.BlockSpec((1, _D), lambda b, j: (0, 0)),
        pl.BlockSpec((1, _D), lambda b, j: (0, 0)),
        pl.BlockSpec((1, _D), lambda b, j: (0, 0)),
    ],
    out_specs=[
        pl.BlockSpec((1, _RB, _D), lambda b, j: (b, j, 0)),
        pl.BlockSpec((1, _RB, _D), lambda b, j: (b, j, 0)),
        pl.BlockSpec((1, _RB, _D), lambda b, j: (b, j, 0)),
    ],
    out_shape=[
        jax.ShapeDtypeStruct((_B, _L, _D), jnp.float32),
        jax.ShapeDtypeStruct((_B, _L, _D), jnp.float32),
        jax.ShapeDtypeStruct((_B, _L, _D), jnp.float32),
    ],
)


def _m_body(c_ref, mn_ref, q_ref, k_ref, o_ref, m_scr):
    nblk = _L // _QB
    kcp = jax.lax.dot_general(
        c_ref[...], k_ref[0].astype(jnp.bfloat16), (((1,), (0,)), ((), ())),
        preferred_element_type=jnp.float32)
    for hh in range(2):
        qh = q_ref[0][:, hh * _DK:(hh + 1) * _DK]
        kh = k_ref[0][:, hh * _DK:(hh + 1) * _DK]
        kch = kcp[:, hh * _DK:(hh + 1) * _DK]
        for j in range(nblk):
            qb = qh[j * _QB:(j + 1) * _QB, :]
            s = _dg(qb, kh, ((1,), (1,)))
            mx = jnp.max(s + mn_ref[j * _QB:(j + 1) * _QB, :], axis=1)
            sm = jnp.sum(qb * kch[j * _QB:(j + 1) * _QB, :], axis=1)
            m_scr[j, :] = mx - sm * (1.0 / _L)
        o_ref[0, hh] = jnp.concatenate(
            [m_scr[j:j + 1, :] for j in range(nblk)], axis=1)


_m_call = pl.pallas_call(
    _m_body,
    grid=(_B, _HP),
    in_specs=[
        pl.BlockSpec((_L, _L), lambda b, p: (0, 0)),
        pl.BlockSpec((_L, _L), lambda b, p: (0, 0)),
        pl.BlockSpec((1, _L, 2 * _DK), lambda b, p: (b, 0, p)),
        pl.BlockSpec((1, _L, 2 * _DK), lambda b, p: (b, 0, p)),
    ],
    out_specs=pl.BlockSpec((1, 2, 1, _L), lambda b, p: (b, p, 0, 0)),
    out_shape=jax.ShapeDtypeStruct((_B, _H, 1, _L), jnp.float32),
    scratch_shapes=[
        pltpu.VMEM((_L // _QB, _QB), jnp.float32),
    ],
)


def _topk_body(m_ref, r_ref):
    m = m_ref[...]
    lanes = jax.lax.broadcasted_iota(jnp.int32, (_B * _H, _L), 1)
    rank = jnp.full((_B * _H, _L), -1, jnp.int32)
    big = jnp.int32(2 ** 30)
    for t in range(_U):
        rmax = jnp.max(m, axis=1, keepdims=True)
        cand = jnp.where(m >= rmax, lanes, big)
        pos = jnp.min(cand, axis=1, keepdims=True)
        oh = lanes == pos
        rank = jnp.where(oh, jnp.int32(t), rank)
        m = jnp.where(oh, -jnp.inf, m)
    r_ref[...] = rank


_topk_call = pl.pallas_call(
    _topk_body,
    out_shape=jax.ShapeDtypeStruct((_B * _H, _L), jnp.int32),
)


def _apply_body(r_ref, q_ref, k_ref, v_ref, o_ref):
    outs = []
    riota = jax.lax.broadcasted_iota(jnp.int32, (_UPAD, _L), 0)
    for hh in range(2):
        rank_flat = r_ref[0, hh]
        q = q_ref[0][:, hh * _DK:(hh + 1) * _DK]
        k = k_ref[0][:, hh * _DK:(hh + 1) * _DK]
        v = v_ref[0][:, hh * _DK:(hh + 1) * _DK]
        p_mat = (riota == rank_flat).astype(jnp.float32)
        qred = _dg(p_mat, q, ((1,), (0,)))
        scores = _dg(qred, k, ((1,), (1,))) * (1.0 / np.sqrt(_DK))
        smax = jnp.max(scores, axis=1, keepdims=True)
        e = jnp.exp(scores - smax)
        attn = e / jnp.sum(e, axis=1, keepdims=True)
        upd = _dg(attn, v, ((1,), (0,)))
        sel_col = _dg(p_mat, jnp.ones((_UPAD, 1), jnp.float32),
                      ((0,), (0,)))
        upd_rows = _dg(p_mat, upd, ((0,), (0,)))
        vmean = jnp.mean(v, axis=0, keepdims=True)
        outs.append(jnp.broadcast_to(vmean, (_L, _DK)) * (1.0 - sel_col)
                    + upd_rows)
    o_ref[0] = jnp.concatenate(outs, axis=1)


_apply_call = pl.pallas_call(
    _apply_body,
    grid=(_B, _HP),
    in_specs=[
        pl.BlockSpec((1, 2, 1, _L), lambda b, p: (b, p, 0, 0)),
        pl.BlockSpec((1, _L, 2 * _DK), lambda b, p: (b, 0, p)),
        pl.BlockSpec((1, _L, 2 * _DK), lambda b, p: (b, 0, p)),
        pl.BlockSpec((1, _L, 2 * _DK), lambda b, p: (b, 0, p)),
    ],
    out_specs=pl.BlockSpec((1, _L, 2 * _DK), lambda b, p: (b, 0, p)),
    out_shape=jax.ShapeDtypeStruct((_B, _L, _D), jnp.float32),
)


def _oproj_body(ctx_ref, enc_ref, w_ref, b_ref, g_ref, bb_ref, o_ref):
    t = _dot(ctx_ref[0], w_ref[...]) + b_ref[...] + enc_ref[0]
    o_ref[0] = _ln(t, g_ref[...], bb_ref[...])


_oproj_call = pl.pallas_call(
    _oproj_body,
    grid=(_B, _L // _RB),
    in_specs=[
        pl.BlockSpec((1, _RB, _D), lambda b, j: (b, j, 0)),
        pl.BlockSpec((1, _RB, _D), lambda b, j: (b, j, 0)),
        pl.BlockSpec((_D, _D), lambda b, j: (0, 0)),
        pl.BlockSpec((1, _D), lambda b, j: (0, 0)),
        pl.BlockSpec((1, _D), lambda b, j: (0, 0)),
        pl.BlockSpec((1, _D), lambda b, j: (0, 0)),
    ],
    out_specs=pl.BlockSpec((1, _RB, _D), lambda b, j: (b, j, 0)),
    out_shape=jax.ShapeDtypeStruct((_B, _L, _D), jnp.float32),
)


def _ffn1_body(x_ref, w_ref, b_ref, o_ref):
    y = _dot(x_ref[0], w_ref[...]) + b_ref[...]
    o_ref[0] = 0.5 * y * (1.0 + jax.lax.erf(y * np.float32(1.0 / np.sqrt(2.0))))


_ffn1_call = pl.pallas_call(
    _ffn1_body,
    grid=(_B, _L // _RB),
    in_specs=[
        pl.BlockSpec((1, _RB, _D), lambda b, j: (b, j, 0)),
        pl.BlockSpec((_D, _DFF), lambda b, j: (0, 0)),
        pl.BlockSpec((1, _DFF), lambda b, j: (0, 0)),
    ],
    out_specs=pl.BlockSpec((1, _RB, _DFF), lambda b, j: (b, j, 0)),
    out_shape=jax.ShapeDtypeStruct((_B, _L, _DFF), jnp.float32),
)


def _ffn2_body(y_ref, xres_ref, w_ref, b_ref, g_ref, bb_ref, o_ref):
    t = _dot(y_ref[0], w_ref[...]) + b_ref[...] + xres_ref[0]
    o_ref[0] = _ln(t, g_ref[...], bb_ref[...])


_ffn2_call = pl.pallas_call(
    _ffn2_body,
    grid=(_B, _L // _RB),
    in_specs=[
        pl.BlockSpec((1, _RB, _DFF), lambda b, j: (b, j, 0)),
        pl.BlockSpec((1, _RB, _D), lambda b, j: (b, j, 0)),
        pl.BlockSpec((_DFF, _D), lambda b, j: (0, 0)),
        pl.BlockSpec((1, _D), lambda b, j: (0, 0)),
        pl.BlockSpec((1, _D), lambda b, j: (0, 0)),
        pl.BlockSpec((1, _D), lambda b, j: (0, 0)),
    ],
    out_specs=pl.BlockSpec((1, _RB, _D), lambda b, j: (b, j, 0)),
    out_shape=jax.ShapeDtypeStruct((_B, _L, _D), jnp.float32),
)


def _ffn2f_body(y_ref, xres_ref, w_ref, b_ref, g_ref, bb_ref, fg_ref,
                fb_ref, mark_ref, o_ref):
    t = _dot(y_ref[0], w_ref[...]) + b_ref[...] + xres_ref[0]
    x1 = _ln(t, g_ref[...], bb_ref[...])
    o_ref[0] = _ln(x1, fg_ref[...], fb_ref[...]) * mark_ref[0]


_ffn2f_call = pl.pallas_call(
    _ffn2f_body,
    grid=(_B, _L // _RB),
    in_specs=[
        pl.BlockSpec((1, _RB, _DFF), lambda b, j: (b, j, 0)),
        pl.BlockSpec((1, _RB, _D), lambda b, j: (b, j, 0)),
        pl.BlockSpec((_DFF, _D), lambda b, j: (0, 0)),
        pl.BlockSpec((1, _D), lambda b, j: (0, 0)),
        pl.BlockSpec((1, _D), lambda b, j: (0, 0)),
        pl.BlockSpec((1, _D), lambda b, j: (0, 0)),
        pl.BlockSpec((1, _D), lambda b, j: (0, 0)),
        pl.BlockSpec((1, _D), lambda b, j: (0, 0)),
        pl.BlockSpec((1, _RB, 1), lambda b, j: (b, j, 0)),
    ],
    out_specs=pl.BlockSpec((1, _RB, _D), lambda b, j: (b, j, 0)),
    out_shape=jax.ShapeDtypeStruct((_B, _L, _D), jnp.float32),
)


def _final_body(x_ref, mark_ref, g_ref, b_ref, o_ref):
    o_ref[0] = _ln(x_ref[0], g_ref[...], b_ref[...]) * mark_ref[0]


_final_call = pl.pallas_call(
    _final_body,
    grid=(_B, _L // _RB),
    in_specs=[
        pl.BlockSpec((1, _RB, _D), lambda b, j: (b, j, 0)),
        pl.BlockSpec((1, _RB, 1), lambda b, j: (b, j, 0)),
        pl.BlockSpec((1, _D), lambda b, j: (0, 0)),
        pl.BlockSpec((1, _D), lambda b, j: (0, 0)),
    ],
    out_specs=pl.BlockSpec((1, _RB, _D), lambda b, j: (b, j, 0)),
    out_shape=jax.ShapeDtypeStruct((_B, _L, _D), jnp.float32),
)


def kernel(x_enc, x_mark_enc, token_w, Wq, bq, Wk, bk, Wv, bv, Wo, bo,
           W1, b1, W2, b2, n1g, n1b, n2g, n2b, fg, fb):
    xprev = jnp.concatenate([x_enc[:, -1:, :], x_enc[:, :-1, :]], axis=1)
    xnext = jnp.concatenate([x_enc[:, 1:, :], x_enc[:, :1, :]], axis=1)
    xcat = jnp.concatenate([xprev, x_enc, xnext], axis=-1)
    wcat = jnp.transpose(token_w, (2, 1, 0)).reshape(3 * _ENC_IN, _D)
    enc = _embed_call(xcat, wcat, jnp.asarray(_POS_PE))
    mark = x_mark_enc[:, :, None]
    for i in range(_LAYERS):
        cmat = jnp.asarray(_CMATS[i], dtype=jnp.bfloat16)
        q, k, v = _qkv_call(enc, Wq[i], Wk[i], Wv[i],
                            bq[i][None], bk[i][None], bv[i][None])
        m = _m_call(cmat, jnp.asarray(_MASKNEG[i]), q, k)
        rank = _topk_call(m.reshape(_B * _H, _L))
        ctx = _apply_call(rank.reshape(_B, _H, 1, _L), q, k, v)
        xres = _oproj_call(ctx, enc, Wo[i], bo[i][None],
                           n1g[i][None], n1b[i][None])
        y1 = _ffn1_call(xres, W1[i], b1[i][None])
        if i < _LAYERS - 1:
            enc = _ffn2_call(y1, xres, W2[i], b2[i][None],
                             n2g[i][None], n2b[i][None])
        else:
            enc = _ffn2f_call(y1, xres, W2[i], b2[i][None],
                              n2g[i][None], n2b[i][None],
                              fg[None], fb[None], mark)
    return enc.reshape(_B, _L * _D)


# speedup vs baseline: 9.7210x; 1.0145x over previous
import numpy as np
import jax
import jax.numpy as jnp
from jax.experimental import pallas as pl
from jax.experimental.pallas import tpu as pltpu

_B, _L, _ENC_IN = 2, 2048, 32
_D, _H, _DFF, _LAYERS = 768, 12, 1024, 2
_DK = _D // _H
_HP = _H // 2
_U = 40
_UPAD = 64
_RB = 512
_QB = 256
_PREC = jax.lax.Precision.DEFAULT


def _pos_embedding(length, d_model):
    pe = np.zeros((length, d_model), dtype=np.float32)
    position = np.arange(length, dtype=np.float32)[:, None]
    div_term = np.exp(
        np.arange(0, d_model, 2, dtype=np.float32) * -(np.log(10000.0) / d_model))
    pe[:, 0::2] = np.sin(position * div_term)
    pe[:, 1::2] = np.cos(position * div_term)
    return pe


_POS_PE = _pos_embedding(_L, _D)


def _sample_count_mats():
    base = jax.random.key(42)
    mats = []
    for i in range(_LAYERS):
        lk = jax.random.fold_in(base, i)
        idx = np.asarray(jax.random.randint(lk, (_L, _U), 0, _L))
        c = np.zeros((_L, _L), np.float32)
        np.add.at(c, (np.arange(_L)[:, None], idx), 1.0)
        mats.append(c)
    return mats


_CMATS = _sample_count_mats()
_MASKNEG = [np.where(c > 0, np.float32(0), np.float32(-np.inf))
            for c in _CMATS]


def _ln(t, g, b):
    mu = jnp.mean(t, axis=-1, keepdims=True)
    var = jnp.mean((t - mu) ** 2, axis=-1, keepdims=True)
    return (t - mu) / jnp.sqrt(var + 1e-5) * g + b


def _dot(a, b):
    return jnp.dot(a.astype(jnp.bfloat16), b.astype(jnp.bfloat16),
                   preferred_element_type=jnp.float32, precision=_PREC)


def _dg(a, b, dims):
    return jax.lax.dot_general(a.astype(jnp.bfloat16), b.astype(jnp.bfloat16),
                               (dims, ((), ())),
                               preferred_element_type=jnp.float32,
                               precision=_PREC)


def _embed_body(x_ref, w_ref, pe_ref, o_ref):
    o_ref[0] = _dot(x_ref[0], w_ref[...]) + pe_ref[...]


_embed_call = pl.pallas_call(
    _embed_body,
    grid=(_B, _L // _RB),
    in_specs=[
        pl.BlockSpec((1, _RB, 3 * _ENC_IN), lambda b, j: (b, j, 0)),
        pl.BlockSpec((3 * _ENC_IN, _D), lambda b, j: (0, 0)),
        pl.BlockSpec((_RB, _D), lambda b, j: (j, 0)),
    ],
    out_specs=pl.BlockSpec((1, _RB, _D), lambda b, j: (b, j, 0)),
    out_shape=jax.ShapeDtypeStruct((_B, _L, _D), jnp.float32),
)


def _qkv_body(x_ref, wq_ref, wk_ref, wv_ref, bq_ref, bk_ref, bv_ref,
              q_ref, k_ref, v_ref):
    x = x_ref[0]
    q_ref[0] = _dot(x, wq_ref[...]) + bq_ref[...]
    k_ref[0] = _dot(x, wk_ref[...]) + bk_ref[...]
    v_ref[0] = _dot(x, wv_ref[...]) + bv_ref[...]


_qkv_call = pl.pallas_call(
    _qkv_body,
    grid=(_B, _L // _RB),
    in_specs=[
        pl.BlockSpec((1, _RB, _D), lambda b, j: (b, j, 0)),
        pl.BlockSpec((_D, _D), lambda b, j: (0, 0)),
        pl.BlockSpec((_D, _D), lambda b, j: (0, 0)),
        pl.BlockSpec((_D, _D), lambda b, j: (0, 0)),
        pl.BlockSpec((1, _D), lambda b, j: (0, 0)),
        pl.BlockSpec((1, _D), lambda b, j: (0, 0)),
        pl.BlockSpec((1, _D), lambda b, j: (0, 0)),
    ],
    out_specs=[
        pl.BlockSpec((1, _RB, _D), lambda b, j: (b, j, 0)),
        pl.BlockSpec((1, _RB, _D), lambda b, j: (b, j, 0)),
        pl.BlockSpec((1, _RB, _D), lambda b, j: (b, j, 0)),
    ],
    out_shape=[
        jax.ShapeDtypeStruct((_B, _L, _D), jnp.float32),
        jax.ShapeDtypeStruct((_B, _L, _D), jnp.float32),
        jax.ShapeDtypeStruct((_B, _L, _D), jnp.float32),
    ],
)


def _m_body(c_ref, mn_ref, q_ref, k_ref, o_ref, m_scr):
    nblk = _L // _QB
    kcp = jax.lax.dot_general(
        c_ref[...], k_ref[0].astype(jnp.bfloat16), (((1,), (0,)), ((), ())),
        preferred_element_type=jnp.float32)
    for hh in range(2):
        qh = q_ref[0][:, hh * _DK:(hh + 1) * _DK]
        kh = k_ref[0][:, hh * _DK:(hh + 1) * _DK]
        kch = kcp[:, hh * _DK:(hh + 1) * _DK]
        for j in range(nblk):
            qb = qh[j * _QB:(j + 1) * _QB, :]
            s = _dg(qb, kh, ((1,), (1,)))
            mx = jnp.max(s + mn_ref[j * _QB:(j + 1) * _QB, :], axis=1)
            sm = jnp.sum(qb * kch[j * _QB:(j + 1) * _QB, :], axis=1)
            m_scr[j, :] = mx - sm * (1.0 / _L)
        o_ref[0, hh] = jnp.concatenate(
            [m_scr[j:j + 1, :] for j in range(nblk)], axis=1)


_m_call = pl.pallas_call(
    _m_body,
    grid=(_B, _HP),
    in_specs=[
        pl.BlockSpec((_L, _L), lambda b, p: (0, 0)),
        pl.BlockSpec((_L, _L), lambda b, p: (0, 0)),
        pl.BlockSpec((1, _L, 2 * _DK), lambda b, p: (b, 0, p)),
        pl.BlockSpec((1, _L, 2 * _DK), lambda b, p: (b, 0, p)),
    ],
    out_specs=pl.BlockSpec((1, 2, 1, _L), lambda b, p: (b, p, 0, 0)),
    out_shape=jax.ShapeDtypeStruct((_B, _H, 1, _L), jnp.float32),
    scratch_shapes=[
        pltpu.VMEM((_L // _QB, _QB), jnp.float32),
    ],
)


def _topk_body(m_ref, r_ref):
    m = m_ref[...]
    lanes = jax.lax.broadcasted_iota(jnp.int32, (_B * _H, _L), 1)
    rank = jnp.full((_B * _H, _L), -1, jnp.int32)
    big = jnp.int32(2 ** 30)
    for t in range(_U):
        rmax = jnp.max(m, axis=1, keepdims=True)
        cand = jnp.where(m >= rmax, lanes, big)
        pos = jnp.min(cand, axis=1, keepdims=True)
        oh = lanes == pos
        rank = jnp.where(oh, jnp.int32(t), rank)
        m = jnp.where(oh, -jnp.inf, m)
    r_ref[...] = rank


_topk_call = pl.pallas_call(
    _topk_body,
    out_shape=jax.ShapeDtypeStruct((_B * _H, _L), jnp.int32),
)


def _apply_body(r_ref, q_ref, k_ref, v_ref, o_ref):
    outs = []
    riota = jax.lax.broadcasted_iota(jnp.int32, (_UPAD, _L), 0)
    for hh in range(2):
        rank_flat = r_ref[0, hh]
        q = q_ref[0][:, hh * _DK:(hh + 1) * _DK]
        k = k_ref[0][:, hh * _DK:(hh + 1) * _DK]
        v = v_ref[0][:, hh * _DK:(hh + 1) * _DK]
        p_mat = (riota == rank_flat).astype(jnp.float32)
        qred = _dg(p_mat, q, ((1,), (0,)))
        scores = _dg(qred, k, ((1,), (1,))) * (1.0 / np.sqrt(_DK))
        smax = jnp.max(scores, axis=1, keepdims=True)
        e = jnp.exp(scores - smax)
        attn = e / jnp.sum(e, axis=1, keepdims=True)
        upd = _dg(attn, v, ((1,), (0,)))
        sel_col = _dg(p_mat, jnp.ones((_UPAD, 1), jnp.float32),
                      ((0,), (0,)))
        upd_rows = _dg(p_mat, upd, ((0,), (0,)))
        vmean = jnp.mean(v, axis=0, keepdims=True)
        outs.append(jnp.broadcast_to(vmean, (_L, _DK)) * (1.0 - sel_col)
                    + upd_rows)
    o_ref[0] = jnp.concatenate(outs, axis=1)


_apply_call = pl.pallas_call(
    _apply_body,
    grid=(_B, _HP),
    in_specs=[
        pl.BlockSpec((1, 2, 1, _L), lambda b, p: (b, p, 0, 0)),
        pl.BlockSpec((1, _L, 2 * _DK), lambda b, p: (b, 0, p)),
        pl.BlockSpec((1, _L, 2 * _DK), lambda b, p: (b, 0, p)),
        pl.BlockSpec((1, _L, 2 * _DK), lambda b, p: (b, 0, p)),
    ],
    out_specs=pl.BlockSpec((1, _L, 2 * _DK), lambda b, p: (b, 0, p)),
    out_shape=jax.ShapeDtypeStruct((_B, _L, _D), jnp.float32),
)


def _oproj_body(ctx_ref, enc_ref, w_ref, b_ref, g_ref, bb_ref, o_ref):
    t = _dot(ctx_ref[0], w_ref[...]) + b_ref[...] + enc_ref[0]
    o_ref[0] = _ln(t, g_ref[...], bb_ref[...])


_oproj_call = pl.pallas_call(
    _oproj_body,
    grid=(_B, _L // _RB),
    in_specs=[
        pl.BlockSpec((1, _RB, _D), lambda b, j: (b, j, 0)),
        pl.BlockSpec((1, _RB, _D), lambda b, j: (b, j, 0)),
        pl.BlockSpec((_D, _D), lambda b, j: (0, 0)),
        pl.BlockSpec((1, _D), lambda b, j: (0, 0)),
        pl.BlockSpec((1, _D), lambda b, j: (0, 0)),
        pl.BlockSpec((1, _D), lambda b, j: (0, 0)),
    ],
    out_specs=pl.BlockSpec((1, _RB, _D), lambda b, j: (b, j, 0)),
    out_shape=jax.ShapeDtypeStruct((_B, _L, _D), jnp.float32),
)


def _ffn1_body(x_ref, w_ref, b_ref, o_ref):
    y = _dot(x_ref[0], w_ref[...]) + b_ref[...]
    o_ref[0] = 0.5 * y * (1.0 + jax.lax.erf(y * np.float32(1.0 / np.sqrt(2.0))))


_ffn1_call = pl.pallas_call(
    _ffn1_body,
    grid=(_B, _L // _RB),
    in_specs=[
        pl.BlockSpec((1, _RB, _D), lambda b, j: (b, j, 0)),
        pl.BlockSpec((_D, _DFF), lambda b, j: (0, 0)),
        pl.BlockSpec((1, _DFF), lambda b, j: (0, 0)),
    ],
    out_specs=pl.BlockSpec((1, _RB, _DFF), lambda b, j: (b, j, 0)),
    out_shape=jax.ShapeDtypeStruct((_B, _L, _DFF), jnp.float32),
)


def _ffn2_body(y_ref, xres_ref, w_ref, b_ref, g_ref, bb_ref, o_ref):
    t = _dot(y_ref[0], w_ref[...]) + b_ref[...] + xres_ref[0]
    o_ref[0] = _ln(t, g_ref[...], bb_ref[...])


_ffn2_call = pl.pallas_call(
    _ffn2_body,
    grid=(_B, _L // _RB),
    in_specs=[
        pl.BlockSpec((1, _RB, _DFF), lambda b, j: (b, j, 0)),
        pl.BlockSpec((1, _RB, _D), lambda b, j: (b, j, 0)),
        pl.BlockSpec((_DFF, _D), lambda b, j: (0, 0)),
        pl.BlockSpec((1, _D), lambda b, j: (0, 0)),
        pl.BlockSpec((1, _D), lambda b, j: (0, 0)),
        pl.BlockSpec((1, _D), lambda b, j: (0, 0)),
    ],
    out_specs=pl.BlockSpec((1, _RB, _D), lambda b, j: (b, j, 0)),
    out_shape=jax.ShapeDtypeStruct((_B, _L, _D), jnp.float32),
)


def _ffn2f_body(y_ref, xres_ref, w_ref, b_ref, g_ref, bb_ref, fg_ref,
                fb_ref, mark_ref, o_ref):
    t = _dot(y_ref[0], w_ref[...]) + b_ref[...] + xres_ref[0]
    x1 = _ln(t, g_ref[...], bb_ref[...])
    o_ref[0] = _ln(x1, fg_ref[...], fb_ref[...]) * mark_ref[0]


_ffn2f_call = pl.pallas_call(
    _ffn2f_body,
    grid=(_B, _L // _RB),
    in_specs=[
        pl.BlockSpec((1, _RB, _DFF), lambda b, j: (b, j, 0)),
        pl.BlockSpec((1, _RB, _D), lambda b, j: (b, j, 0)),
        pl.BlockSpec((_DFF, _D), lambda b, j: (0, 0)),
        pl.BlockSpec((1, _D), lambda b, j: (0, 0)),
        pl.BlockSpec((1, _D), lambda b, j: (0, 0)),
        pl.BlockSpec((1, _D), lambda b, j: (0, 0)),
        pl.BlockSpec((1, _D), lambda b, j: (0, 0)),
        pl.BlockSpec((1, _D), lambda b, j: (0, 0)),
        pl.BlockSpec((1, _RB, 1), lambda b, j: (b, j, 0)),
    ],
    out_specs=pl.BlockSpec((1, _RB, _D), lambda b, j: (b, j, 0)),
    out_shape=jax.ShapeDtypeStruct((_B, _L, _D), jnp.float32),
)


def _final_body(x_ref, mark_ref, g_ref, b_ref, o_ref):
    o_ref[0] = _ln(x_ref[0], g_ref[...], b_ref[...]) * mark_ref[0]


_final_call = pl.pallas_call(
    _final_body,
    grid=(_B, _L // _RB),
    in_specs=[
        pl.BlockSpec((1, _RB, _D), lambda b, j: (b, j, 0)),
        pl.BlockSpec((1, _RB, 1), lambda b, j: (b, j, 0)),
        pl.BlockSpec((1, _D), lambda b, j: (0, 0)),
        pl.BlockSpec((1, _D), lambda b, j: (0, 0)),
    ],
    out_specs=pl.BlockSpec((1, _RB, _D), lambda b, j: (b, j, 0)),
    out_shape=jax.ShapeDtypeStruct((_B, _L, _D), jnp.float32),
)


def kernel(x_enc, x_mark_enc, token_w, Wq, bq, Wk, bk, Wv, bv, Wo, bo,
           W1, b1, W2, b2, n1g, n1b, n2g, n2b, fg, fb):
    xprev = jnp.concatenate([x_enc[:, -1:, :], x_enc[:, :-1, :]], axis=1)
    xnext = jnp.concatenate([x_enc[:, 1:, :], x_enc[:, :1, :]], axis=1)
    xcat = jnp.concatenate([xprev, x_enc, xnext], axis=-1)
    wcat = jnp.transpose(token_w, (2, 1, 0)).reshape(3 * _ENC_IN, _D)
    enc = _embed_call(xcat, wcat, jnp.asarray(_POS_PE))
    mark = x_mark_enc[:, :, None]
    for i in range(_LAYERS):
        cmat = jnp.asarray(_CMATS[i], dtype=jnp.bfloat16)
        q, k, v = _qkv_call(enc, Wq[i], Wk[i], Wv[i],
                            bq[i][None], bk[i][None], bv[i][None])
        m = _m_call(cmat, jnp.asarray(_MASKNEG[i]), q, k)
        rank = _topk_call(m.reshape(_B * _H, _L))
        ctx = _apply_call(rank.reshape(_B, _H, 1, _L), q, k, v)
        xres = _oproj_call(ctx, enc, Wo[i], bo[i][None],
                           n1g[i][None], n1b[i][None])
        y1 = _ffn1_call(xres, W1[i], b1[i][None])
        if i < _LAYERS - 1:
            enc = _ffn2_call(y1, xres, W2[i], b2[i][None],
                             n2g[i][None], n2b[i][None])
        else:
            enc = _ffn2f_call(y1, xres, W2[i], b2[i][None],
                              n2g[i][None], n2b[i][None],
                              fg[None], fb[None], mark)
    return enc.reshape(_B, _L * _D)
